# baseline (speedup 1.0000x reference)
import functools

import jax
import jax.numpy as jnp
from jax import lax
from jax.experimental import pallas as pl
from jax.experimental.pallas import tpu as pltpu

_TAU = 0.1
_S = 32


def _nce_body(nblk, bh, w, emb_ref, seg_ref, segf_ref, e0_ref, e1_ref,
              t_ref, s_ref, cache_ref, sums_ref, cnt_ref, means_ref, nam_ref,
              accT_ref, accE_ref):
    phase = pl.program_id(0)
    i = pl.program_id(1)

    @pl.when(jnp.logical_and(phase == 0, i == 0))
    def _init():
        sums_ref[...] = jnp.zeros_like(sums_ref)
        cnt_ref[...] = jnp.zeros_like(cnt_ref)
        accT_ref[...] = jnp.zeros_like(accT_ref)
        accE_ref[...] = jnp.zeros_like(accE_ref)

    @pl.when(phase == 0)
    def _pass1():
        eb = emb_ref[0]
        cache_ref[i] = eb
        ebf = eb.reshape(eb.shape[0], bh * w)
        segf = segf_ref[0]
        iota_f = lax.broadcasted_iota(jnp.int32, (_S, bh * w), 0)
        ohf = (iota_f == segf).astype(jnp.float32)
        sums_ref[...] += lax.dot_general(
            ohf, ebf, (((1,), (1,)), ((), ())),
            preferred_element_type=jnp.float32)
        cnt_ref[...] += jnp.sum(ohf.reshape(_S, (bh * w) // 128, 128), axis=1)

    @pl.when(jnp.logical_and(phase == 1, i == 0))
    def _means():
        counts = jnp.sum(cnt_ref[...], axis=1, keepdims=True)
        means = sums_ref[...] / counts
        means_ref[...] = means
        nam_ref[...] = jnp.sqrt(jnp.sum(means * means, axis=1, keepdims=True))

    @pl.when(phase == 1)
    def _pass2():
        seg = seg_ref[...]
        iota_s = lax.broadcasted_iota(jnp.int32, (_S, bh, w), 0)
        oh = (iota_s == seg[None]).astype(jnp.float32)
        eb = cache_ref[i]
        means = means_ref[...]
        dots = lax.dot_general(
            means, eb, (((1,), (0,)), ((), ())),
            preferred_element_type=jnp.float32)
        dot_p = jnp.sum(dots * oh, axis=0)
        na_p = jnp.sum(nam_ref[...][:, :, None] * oh, axis=0)
        nbsq = jnp.sum(eb * eb, axis=0)
        nb_p = jnp.sqrt(nbsq)
        sim = dot_p / (jnp.maximum(na_p * nb_p, 1e-8) * _TAU)
        accT_ref[...] += jnp.sum(sim.reshape(bh // 8, 8, w), axis=0)
        accE_ref[...] += jnp.sum(jnp.exp(sim).reshape(bh // 8, 8, w), axis=0)

    @pl.when(jnp.logical_and(phase == 1, i == nblk - 1))
    def _fin():
        means = means_ref[...]
        na = nam_ref[...]
        e0 = e0_ref[...]
        e1 = e1_ref[...]
        it = lax.broadcasted_iota(jnp.int32, (_S, e0.shape[-1]), 0)
        oh0 = (it == e0).astype(jnp.float32)
        oh1 = (it == e1).astype(jnp.float32)
        cnt_ij = lax.dot_general(
            oh0, oh1, (((1,), (1,)), ((), ())), preferred_element_type=jnp.float32)
        gram = lax.dot_general(
            means, means, (((1,), (1,)), ((), ())), preferred_element_type=jnp.float32)
        na_outer = lax.dot_general(
            na, na, (((1,), (1,)), ((), ())), preferred_element_type=jnp.float32)
        cos_ij = gram / jnp.maximum(na_outer, 1e-8) / _TAU
        edge_s = jnp.sum(cnt_ij * jnp.exp(cos_ij))
        t_ref[0, 0] = jnp.sum(accT_ref[...])
        s_ref[0, 0] = jnp.sum(accE_ref[...]) + edge_s


def kernel(embeddings, sp_seg, edges):
    C = embeddings.shape[1]
    H, W = embeddings.shape[2], embeddings.shape[3]
    BH = 56
    nblk = H // BH
    npix = H * W
    seg = sp_seg.reshape(H, W)
    segf = sp_seg.reshape(nblk, 1, BH * W)
    e0 = edges[0:1, :]
    e1 = edges[1:2, :]

    body = functools.partial(_nce_body, nblk, BH, W)
    t, s = pl.pallas_call(
        body,
        grid=(2, nblk),
        in_specs=[
            pl.BlockSpec((1, C, BH, W), lambda p, i: (0, 0, i * (1 - p), 0)),
            pl.BlockSpec((BH, W), lambda p, i: (i, 0)),
            pl.BlockSpec((1, 1, BH * W), lambda p, i: (i * (1 - p), 0, 0)),
            pl.BlockSpec((1, edges.shape[1]), lambda p, i: (0, 0)),
            pl.BlockSpec((1, edges.shape[1]), lambda p, i: (0, 0)),
        ],
        out_specs=[
            pl.BlockSpec(memory_space=pltpu.SMEM),
            pl.BlockSpec(memory_space=pltpu.SMEM),
        ],
        out_shape=[
            jax.ShapeDtypeStruct((1, 1), jnp.float32),
            jax.ShapeDtypeStruct((1, 1), jnp.float32),
        ],
        scratch_shapes=[
            pltpu.VMEM((nblk, C, BH, W), jnp.float32),
            pltpu.VMEM((_S, C), jnp.float32),
            pltpu.VMEM((_S, 128), jnp.float32),
            pltpu.VMEM((_S, C), jnp.float32),
            pltpu.VMEM((_S, 1), jnp.float32),
            pltpu.VMEM((8, W), jnp.float32),
            pltpu.VMEM((8, W), jnp.float32),
        ],
        compiler_params=pltpu.CompilerParams(
            dimension_semantics=("arbitrary", "arbitrary"),
        ),
    )(embeddings, seg, segf, e0, e1)
    return jnp.log(s[0, 0]) - t[0, 0] / jnp.float32(npix)


# speedup vs baseline: 1.0173x; 1.0173x over previous
import functools

import jax
import jax.numpy as jnp
from jax import lax
from jax.experimental import pallas as pl
from jax.experimental.pallas import tpu as pltpu

_TAU = 0.1
_S = 32


def _nce_body(nblk, bh, w, emb_ref, seg_ref, segf_ref, e0_ref, e1_ref,
              t_ref, s_ref, sums_ref, cnt_ref, means_ref, nam_ref,
              accT_ref, accE_ref):
    phase = pl.program_id(0)
    i = pl.program_id(1)

    @pl.when(jnp.logical_and(phase == 0, i == 0))
    def _init():
        sums_ref[...] = jnp.zeros_like(sums_ref)
        cnt_ref[...] = jnp.zeros_like(cnt_ref)
        accT_ref[...] = jnp.zeros_like(accT_ref)
        accE_ref[...] = jnp.zeros_like(accE_ref)

    @pl.when(phase == 0)
    def _pass1():
        eb = emb_ref[0]
        ebf = eb.reshape(eb.shape[0], bh * w)
        segf = segf_ref[0]
        iota_f = lax.broadcasted_iota(jnp.int32, (_S, bh * w), 0)
        ohf = (iota_f == segf).astype(jnp.float32)
        sums_ref[...] += lax.dot_general(
            ohf, ebf, (((1,), (1,)), ((), ())),
            preferred_element_type=jnp.float32)
        cnt_ref[...] += jnp.sum(ohf.reshape(_S, (bh * w) // 128, 128), axis=1)

    @pl.when(jnp.logical_and(phase == 1, i == 0))
    def _means():
        counts = jnp.sum(cnt_ref[...], axis=1, keepdims=True)
        means = sums_ref[...] / counts
        means_ref[...] = means
        nam_ref[...] = jnp.sqrt(jnp.sum(means * means, axis=1, keepdims=True))

    @pl.when(phase == 1)
    def _pass2():
        seg = seg_ref[...]
        iota_s = lax.broadcasted_iota(jnp.int32, (_S, bh, w), 0)
        oh = (iota_s == seg[None]).astype(jnp.float32)
        eb = emb_ref[0]
        means = means_ref[...]
        dots = lax.dot_general(
            means, eb, (((1,), (0,)), ((), ())),
            preferred_element_type=jnp.float32)
        dot_p = jnp.sum(dots * oh, axis=0)
        na_p = jnp.sum(nam_ref[...][:, :, None] * oh, axis=0)
        nbsq = jnp.sum(eb * eb, axis=0)
        nb_p = jnp.sqrt(nbsq)
        sim = dot_p / (jnp.maximum(na_p * nb_p, 1e-8) * _TAU)
        accT_ref[...] += jnp.sum(sim.reshape(bh // 8, 8, w), axis=0)
        accE_ref[...] += jnp.sum(jnp.exp(sim).reshape(bh // 8, 8, w), axis=0)

    @pl.when(jnp.logical_and(phase == 1, i == nblk - 1))
    def _fin():
        means = means_ref[...]
        na = nam_ref[...]
        e0 = e0_ref[...]
        e1 = e1_ref[...]
        it = lax.broadcasted_iota(jnp.int32, (_S, e0.shape[-1]), 0)
        oh0 = (it == e0).astype(jnp.float32)
        oh1 = (it == e1).astype(jnp.float32)
        cnt_ij = lax.dot_general(
            oh0, oh1, (((1,), (1,)), ((), ())), preferred_element_type=jnp.float32)
        gram = lax.dot_general(
            means, means, (((1,), (1,)), ((), ())), preferred_element_type=jnp.float32)
        na_outer = lax.dot_general(
            na, na, (((1,), (1,)), ((), ())), preferred_element_type=jnp.float32)
        cos_ij = gram / jnp.maximum(na_outer, 1e-8) / _TAU
        edge_s = jnp.sum(cnt_ij * jnp.exp(cos_ij))
        t_ref[0, 0] = jnp.sum(accT_ref[...])
        s_ref[0, 0] = jnp.sum(accE_ref[...]) + edge_s


def kernel(embeddings, sp_seg, edges):
    C = embeddings.shape[1]
    H, W = embeddings.shape[2], embeddings.shape[3]
    BH = 56
    nblk = H // BH
    npix = H * W
    seg = sp_seg.reshape(H, W)
    segf = sp_seg.reshape(nblk, 1, BH * W)
    e0 = edges[0:1, :]
    e1 = edges[1:2, :]

    body = functools.partial(_nce_body, nblk, BH, W)
    t, s = pl.pallas_call(
        body,
        grid=(2, nblk),
        in_specs=[
            pl.BlockSpec((1, C, BH, W), lambda p, i: (0, 0, i, 0)),
            pl.BlockSpec((BH, W), lambda p, i: (i, 0)),
            pl.BlockSpec((1, 1, BH * W), lambda p, i: (i * (1 - p), 0, 0)),
            pl.BlockSpec((1, edges.shape[1]), lambda p, i: (0, 0)),
            pl.BlockSpec((1, edges.shape[1]), lambda p, i: (0, 0)),
        ],
        out_specs=[
            pl.BlockSpec(memory_space=pltpu.SMEM),
            pl.BlockSpec(memory_space=pltpu.SMEM),
        ],
        out_shape=[
            jax.ShapeDtypeStruct((1, 1), jnp.float32),
            jax.ShapeDtypeStruct((1, 1), jnp.float32),
        ],
        scratch_shapes=[
            pltpu.VMEM((_S, C), jnp.float32),
            pltpu.VMEM((_S, 128), jnp.float32),
            pltpu.VMEM((_S, C), jnp.float32),
            pltpu.VMEM((_S, 1), jnp.float32),
            pltpu.VMEM((8, W), jnp.float32),
            pltpu.VMEM((8, W), jnp.float32),
        ],
        compiler_params=pltpu.CompilerParams(
            dimension_semantics=("arbitrary", "arbitrary"),
        ),
    )(embeddings, seg, segf, e0, e1)
    return jnp.log(s[0, 0]) - t[0, 0] / jnp.float32(npix)
